# baseline (speedup 1.0000x reference)
import functools

import jax
import jax.numpy as jnp
from jax import lax
from jax.experimental import pallas as pl
from jax.experimental.pallas import tpu as pltpu
from jax.experimental.pallas import tpu_sc as plsc

_LANES = 16
_NC = 2
_NS = 16
_NW = _NC * _NS
_TPAD = 32
_TABS = 64
_CHUNK = 65536


@functools.lru_cache(maxsize=None)
def _make_sc_interp(n, npts):
    per_w = n // _NW
    n_chunks = per_w // _CHUNK
    hi_idx = npts - 2

    mesh = plsc.VectorSubcoreMesh(
        core_axis_name="c", subcore_axis_name="s",
        num_cores=_NC, num_subcores=_NS)

    @functools.partial(
        pl.kernel,
        out_type=jax.ShapeDtypeStruct((n,), jnp.float32),
        mesh=mesh,
        compiler_params=pltpu.CompilerParams(needs_layout_passes=False),
        scratch_types=[
            pltpu.VMEM((_TABS,), jnp.float32),
            pltpu.VMEM((_CHUNK,), jnp.float32),
            pltpu.SemaphoreType.DMA,
            pltpu.SemaphoreType.DMA,
        ],
    )
    def sc_interp(soc_hbm, tabs_hbm, out_hbm, tabs_v, buf, sem_in, sem_out):
        wid = lax.axis_index("s") * _NC + lax.axis_index("c")
        base = wid * per_w

        pltpu.sync_copy(tabs_hbm, tabs_v)
        bv = tabs_v[pl.ds(_TPAD, _LANES)]
        inv = tabs_v[pl.ds(_TPAD + _LANES, _LANES)]

        def do_chunk(c, carry):
            off = base + c * _CHUNK
            pltpu.async_copy(soc_hbm.at[pl.ds(off, _CHUNK)], buf, sem_in).wait()

            def body(i, carry2):
                sl = pl.ds(pl.multiple_of(i * _LANES, _LANES), _LANES)
                x = buf[sl]
                xc = jnp.minimum(jnp.maximum(x, 0.0), 1.0)
                t = xc * inv - bv
                idx = jnp.minimum(t.astype(jnp.int32), hi_idx)
                w = t - idx.astype(jnp.float32)
                v0 = plsc.load_gather(tabs_v, [idx])
                v1 = plsc.load_gather(tabs_v, [idx + 1])
                buf[sl] = v0 + w * (v1 - v0)
                return carry2

            lax.fori_loop(0, _CHUNK // _LANES, body, 0)
            pltpu.async_copy(buf, out_hbm.at[pl.ds(off, _CHUNK)], sem_out).wait()
            return carry

        lax.fori_loop(0, n_chunks, do_chunk, 0)

    return sc_interp


def kernel(soc, soc_table, ocv_table):
    n = soc.shape[0]
    npts = soc_table.shape[0]
    t0 = soc_table[0]
    inv = 1.0 / (soc_table[1] - soc_table[0])
    tabs = jnp.zeros((_TABS,), jnp.float32)
    tabs = tabs.at[:npts].set(ocv_table)
    tabs = tabs.at[_TPAD:_TPAD + _LANES].set(t0 * inv)
    tabs = tabs.at[_TPAD + _LANES:].set(inv)
    return _make_sc_interp(n, npts)(soc, tabs)


# speedup vs baseline: 7.0353x; 7.0353x over previous
import functools

import jax
import jax.numpy as jnp
from jax import lax
from jax.experimental import pallas as pl
from jax.experimental.pallas import tpu as pltpu
from jax.experimental.pallas import tpu_sc as plsc

_LANES = 16
_NC = 2
_NS = 16
_NW = _NC * _NS
_TPAD = 32
_TABS = 64
_CHUNK = 16384
_UNROLL = 8


@functools.lru_cache(maxsize=None)
def _make_sc_interp(n, npts):
    per_w = n // _NW
    n_chunks = per_w // _CHUNK
    n_pairs = n_chunks // 2
    hi_idx = npts - 2

    mesh = plsc.VectorSubcoreMesh(
        core_axis_name="c", subcore_axis_name="s",
        num_cores=_NC, num_subcores=_NS)

    @functools.partial(
        pl.kernel,
        out_type=jax.ShapeDtypeStruct((n,), jnp.float32),
        mesh=mesh,
        compiler_params=pltpu.CompilerParams(needs_layout_passes=False),
        scratch_types=[
            pltpu.VMEM((_TABS,), jnp.float32),
            pltpu.VMEM((_CHUNK,), jnp.float32),
            pltpu.VMEM((_CHUNK,), jnp.float32),
            pltpu.VMEM((_CHUNK,), jnp.float32),
            pltpu.VMEM((_CHUNK,), jnp.float32),
            pltpu.SemaphoreType.DMA,
            pltpu.SemaphoreType.DMA,
            pltpu.SemaphoreType.DMA,
            pltpu.SemaphoreType.DMA,
        ],
    )
    def sc_interp(soc_hbm, tabs_hbm, out_hbm, tabs_v,
                  in0, in1, ot0, ot1, si0, si1, so0, so1):
        wid = lax.axis_index("s") * _NC + lax.axis_index("c")
        base = wid * per_w
        ins, ots = (in0, in1), (ot0, ot1)
        sis, sos = (si0, si1), (so0, so1)

        pltpu.sync_copy(tabs_hbm, tabs_v)
        bv = tabs_v[pl.ds(_TPAD, _LANES)]
        inv = tabs_v[pl.ds(_TPAD + _LANES, _LANES)]

        pltpu.async_copy(soc_hbm.at[pl.ds(base, _CHUNK)], in0, si0)
        pltpu.async_copy(soc_hbm.at[pl.ds(base + _CHUNK, _CHUNK)], in1, si1)

        def do_pair(c2, carry):
            for b in range(2):
                c = c2 * 2 + b
                off = base + c * _CHUNK
                ib, ob = ins[b], ots[b]
                pltpu.make_async_copy(
                    soc_hbm.at[pl.ds(base, _CHUNK)], ib, sis[b]).wait()
                @pl.when(c2 > 0)
                def _():
                    pltpu.make_async_copy(
                        ob, out_hbm.at[pl.ds(base, _CHUNK)], sos[b]).wait()

                @plsc.parallel_loop(0, _CHUNK // _LANES, step=1, unroll=_UNROLL)
                def _(i):
                    sl = pl.ds(pl.multiple_of(i * _LANES, _LANES), _LANES)
                    t = ib[sl] * inv - bv
                    idx = jnp.minimum(t.astype(jnp.int32), hi_idx)
                    w = t - idx.astype(jnp.float32)
                    v0 = plsc.load_gather(tabs_v, [idx])
                    v1 = plsc.load_gather(tabs_v, [idx + 1])
                    ob[sl] = v0 + w * (v1 - v0)

                pltpu.async_copy(ob, out_hbm.at[pl.ds(off, _CHUNK)], sos[b])
                @pl.when(c2 < n_pairs - 1)
                def _():
                    pltpu.async_copy(
                        soc_hbm.at[pl.ds(off + 2 * _CHUNK, _CHUNK)], ib, sis[b])
            return carry

        lax.fori_loop(0, n_pairs, do_pair, 0)
        pltpu.make_async_copy(ot0, out_hbm.at[pl.ds(base, _CHUNK)], so0).wait()
        pltpu.make_async_copy(ot1, out_hbm.at[pl.ds(base, _CHUNK)], so1).wait()

    return sc_interp


def kernel(soc, soc_table, ocv_table):
    n = soc.shape[0]
    npts = soc_table.shape[0]
    t0 = soc_table[0]
    inv = 1.0 / (soc_table[1] - soc_table[0])
    tabs = jnp.zeros((_TABS,), jnp.float32)
    tabs = tabs.at[:npts].set(ocv_table)
    tabs = tabs.at[_TPAD:_TPAD + _LANES].set(t0 * inv)
    tabs = tabs.at[_TPAD + _LANES:].set(inv)
    return _make_sc_interp(n, npts)(soc, tabs)


# speedup vs baseline: 8.4100x; 1.1954x over previous
import functools

import jax
import jax.numpy as jnp
from jax import lax
from jax.experimental import pallas as pl
from jax.experimental.pallas import tpu as pltpu
from jax.experimental.pallas import tpu_sc as plsc

_LANES = 16
_NC = 2
_NS = 16
_NW = _NC * _NS
_TPAD = 32
_TABS = 96
_CHUNK = 16384
_UNROLL = 8


@functools.lru_cache(maxsize=None)
def _make_sc_interp(n, npts):
    per_w = n // _NW
    n_chunks = per_w // _CHUNK
    n_pairs = n_chunks // 2
    hi_idx = npts - 2

    mesh = plsc.VectorSubcoreMesh(
        core_axis_name="c", subcore_axis_name="s",
        num_cores=_NC, num_subcores=_NS)

    @functools.partial(
        pl.kernel,
        out_type=jax.ShapeDtypeStruct((n,), jnp.float32),
        mesh=mesh,
        compiler_params=pltpu.CompilerParams(needs_layout_passes=False),
        scratch_types=[
            pltpu.VMEM((_TPAD,), jnp.float32),
            pltpu.VMEM((_TPAD,), jnp.float32),
            pltpu.VMEM((2 * _LANES,), jnp.float32),
            pltpu.VMEM((_CHUNK,), jnp.float32),
            pltpu.VMEM((_CHUNK,), jnp.float32),
            pltpu.VMEM((_CHUNK,), jnp.float32),
            pltpu.VMEM((_CHUNK,), jnp.float32),
            pltpu.SemaphoreType.DMA,
            pltpu.SemaphoreType.DMA,
            pltpu.SemaphoreType.DMA,
            pltpu.SemaphoreType.DMA,
        ],
    )
    def sc_interp(soc_hbm, tabs_hbm, out_hbm, a_v, b_v, p_v,
                  in0, in1, ot0, ot1, si0, si1, so0, so1):
        wid = lax.axis_index("s") * _NC + lax.axis_index("c")
        base = wid * per_w
        ins, ots = (in0, in1), (ot0, ot1)
        sis, sos = (si0, si1), (so0, so1)

        pltpu.sync_copy(tabs_hbm.at[pl.ds(0, _TPAD)], a_v)
        pltpu.sync_copy(tabs_hbm.at[pl.ds(_TPAD, _TPAD)], b_v)
        pltpu.sync_copy(tabs_hbm.at[pl.ds(2 * _TPAD, 2 * _LANES)], p_v)
        bv = p_v[pl.ds(0, _LANES)]
        inv = p_v[pl.ds(_LANES, _LANES)]

        pltpu.async_copy(soc_hbm.at[pl.ds(base, _CHUNK)], in0, si0)
        pltpu.async_copy(soc_hbm.at[pl.ds(base + _CHUNK, _CHUNK)], in1, si1)

        def do_pair(c2, carry):
            for b in range(2):
                c = c2 * 2 + b
                off = base + c * _CHUNK
                ib, ob = ins[b], ots[b]
                pltpu.make_async_copy(
                    soc_hbm.at[pl.ds(base, _CHUNK)], ib, sis[b]).wait()
                @pl.when(c2 > 0)
                def _():
                    pltpu.make_async_copy(
                        ob, out_hbm.at[pl.ds(base, _CHUNK)], sos[b]).wait()

                @plsc.parallel_loop(0, _CHUNK // _LANES, step=1, unroll=_UNROLL)
                def _(i):
                    sl = pl.ds(pl.multiple_of(i * _LANES, _LANES), _LANES)
                    t = ib[sl] * inv - bv
                    idx = jnp.minimum(t.astype(jnp.int32), hi_idx)
                    av = plsc.load_gather(a_v, [idx])
                    bvv = plsc.load_gather(b_v, [idx])
                    ob[sl] = av + t * bvv

                pltpu.async_copy(ob, out_hbm.at[pl.ds(off, _CHUNK)], sos[b])
                @pl.when(c2 < n_pairs - 1)
                def _():
                    pltpu.async_copy(
                        soc_hbm.at[pl.ds(off + 2 * _CHUNK, _CHUNK)], ib, sis[b])
            return carry

        lax.fori_loop(0, n_pairs, do_pair, 0)
        pltpu.make_async_copy(ot0, out_hbm.at[pl.ds(base, _CHUNK)], so0).wait()
        pltpu.make_async_copy(ot1, out_hbm.at[pl.ds(base, _CHUNK)], so1).wait()

    return sc_interp


def kernel(soc, soc_table, ocv_table):
    n = soc.shape[0]
    npts = soc_table.shape[0]
    t0 = soc_table[0]
    inv = 1.0 / (soc_table[1] - soc_table[0])
    dv = ocv_table[1:] - ocv_table[:-1]
    ks = jnp.arange(npts - 1, dtype=jnp.float32)
    av = ocv_table[:-1] - ks * dv
    tabs = jnp.zeros((_TABS,), jnp.float32)
    tabs = tabs.at[:npts - 1].set(av)
    tabs = tabs.at[_TPAD:_TPAD + npts - 1].set(dv)
    tabs = tabs.at[2 * _TPAD:2 * _TPAD + _LANES].set(t0 * inv)
    tabs = tabs.at[2 * _TPAD + _LANES:].set(inv)
    return _make_sc_interp(n, npts)(soc, tabs)
